# baseline (device time: 126377 ns/iter reference)
import jax
import jax.numpy as jnp
from jax import lax
from jax.experimental import pallas as pl
from jax.experimental.pallas import tpu as pltpu

N_DEV = 4
SQ = 1024
SKV = 1024
D = 1024
HQ_LOCAL = 8
DH = 128
SCALE = 0.08838834764831843
CHUNK = SQ // N_DEV
NEG = -1e9


def _body(x_ref, wq_ref, k_ref, v_ref, wo_ref, out_ref,
          acc_ref, rs_buf, rs_send_sems, rs_recv_sems,
          ag_send_sems, ag_recv_sems):
    my = lax.axis_index("i")
    left = lax.rem(my + N_DEV - 1, N_DEV)
    right = lax.rem(my + 1, N_DEV)

    barrier = pltpu.get_barrier_semaphore()
    for nbr in (left, right):
        pl.semaphore_signal(barrier, inc=1, device_id=(nbr,),
                            device_id_type=pl.DeviceIdType.MESH)
    pl.semaphore_wait(barrier, 2)

    q = jnp.dot(x_ref[...], wq_ref[...], preferred_element_type=jnp.float32)

    qb = lax.broadcasted_iota(jnp.int32, (SQ, SKV), 0) // 64
    kb = lax.broadcasted_iota(jnp.int32, (SQ, SKV), 1) // 64
    mask = (qb == kb) | (kb == 0) | (lax.rem(qb + kb, 3) == 0)

    for h in range(HQ_LOCAL):
        qh = q[:, h * DH:(h + 1) * DH]
        s = lax.dot_general(qh, k_ref[h], (((1,), (1,)), ((), ())),
                            preferred_element_type=jnp.float32) * SCALE
        s = jnp.where(mask, s, NEG)
        m = jnp.max(s, axis=-1, keepdims=True)
        w = jnp.exp(s - m)
        w = w / jnp.sum(w, axis=-1, keepdims=True)
        ctx_h = jnp.dot(w, v_ref[h], preferred_element_type=jnp.float32)
        ph = jnp.dot(ctx_h, wo_ref[h * DH:(h + 1) * DH, :],
                     preferred_element_type=jnp.float32)
        if h == 0:
            acc_ref[...] = ph
        else:
            acc_ref[...] = acc_ref[...] + ph

    for h in range(N_DEV - 1):
        s_idx = lax.rem(my - h + N_DEV, N_DEV)
        r_idx = lax.rem(my - h - 1 + N_DEV, N_DEV)
        rdma = pltpu.make_async_remote_copy(
            src_ref=acc_ref.at[pl.ds(s_idx * CHUNK, CHUNK)],
            dst_ref=rs_buf.at[h],
            send_sem=rs_send_sems.at[h],
            recv_sem=rs_recv_sems.at[h],
            device_id=(right,),
            device_id_type=pl.DeviceIdType.MESH,
        )
        rdma.start()
        rdma.wait()
        acc_ref[pl.ds(r_idx * CHUNK, CHUNK), :] = (
            acc_ref[pl.ds(r_idx * CHUNK, CHUNK), :] + rs_buf[h]
        )

    own = lax.rem(my + 1, N_DEV)
    out_ref[pl.ds(own * CHUNK, CHUNK), :] = acc_ref[pl.ds(own * CHUNK, CHUNK), :]
    for h in range(N_DEV - 1):
        g = lax.rem(own - h + N_DEV, N_DEV)
        rdma = pltpu.make_async_remote_copy(
            src_ref=out_ref.at[pl.ds(g * CHUNK, CHUNK)],
            dst_ref=out_ref.at[pl.ds(g * CHUNK, CHUNK)],
            send_sem=ag_send_sems.at[h],
            recv_sem=ag_recv_sems.at[h],
            device_id=(right,),
            device_id_type=pl.DeviceIdType.MESH,
        )
        rdma.start()
        rdma.wait()


def kernel(x, Wq, K_ext, V_ext, Wo):
    my = lax.axis_index("i")
    x2 = x.reshape(SQ, D)
    k_loc = jnp.transpose(
        lax.dynamic_slice_in_dim(K_ext[0], my * HQ_LOCAL, HQ_LOCAL, axis=1),
        (1, 0, 2))
    v_loc = jnp.transpose(
        lax.dynamic_slice_in_dim(V_ext[0], my * HQ_LOCAL, HQ_LOCAL, axis=1),
        (1, 0, 2))

    out = pl.pallas_call(
        _body,
        out_shape=jax.ShapeDtypeStruct((SQ, D), jnp.float32),
        in_specs=[pl.BlockSpec(memory_space=pltpu.VMEM)] * 5,
        out_specs=pl.BlockSpec(memory_space=pltpu.VMEM),
        scratch_shapes=[
            pltpu.VMEM((SQ, D), jnp.float32),
            pltpu.VMEM((N_DEV - 1, CHUNK, D), jnp.float32),
            pltpu.SemaphoreType.DMA((N_DEV - 1,)),
            pltpu.SemaphoreType.DMA((N_DEV - 1,)),
            pltpu.SemaphoreType.DMA((N_DEV - 1,)),
            pltpu.SemaphoreType.DMA((N_DEV - 1,)),
        ],
        compiler_params=pltpu.CompilerParams(collective_id=0),
    )(x2, Wq, k_loc, v_loc, Wo)
    return out.reshape(1, SQ, D)
